# Initial kernel scaffold; baseline (speedup 1.0000x reference)
import jax
import jax.numpy as jnp
from jax.experimental import pallas as pl


def kernel(x, edge_index, W1_l, W1_r, b1, gamma, beta, W2_l, W2_r, b2):
    raise NotImplementedError("write your pallas kernel here")


# speedup vs baseline: 6.3973x; 6.3973x over previous
import functools

import jax
import jax.numpy as jnp
from jax import lax
from jax.experimental import pallas as pl
from jax.experimental.pallas import tpu as pltpu
from jax.experimental.pallas import tpu_sc as plsc

N = 10000
E = 160000
D = 256
HID = 256
NCLS = 40
QW = 64
NQ = D // QW
TPAD = 64

NC, NS = 2, 16
W_IDX = 100
ROWS_A = 4
CHUNK_A = W_IDX * ROWS_A
ITERS_A = E // NS // CHUNK_A
ROWS_B = 10
CHUNK_B = W_IDX * ROWS_B
ITERS_B = E // (NC * NS) // CHUNK_B
RPS = 1000
NSI = N // RPS

_mesh = plsc.VectorSubcoreMesh(core_axis_name="c", subcore_axis_name="s")


@functools.partial(
    pl.kernel,
    out_type=(
        jax.ShapeDtypeStruct((NQ, N, QW), jnp.float32),
        jax.ShapeDtypeStruct((N, 8), jnp.float32),
    ),
    mesh=_mesh,
    scratch_types=(
        pltpu.VMEM((ROWS_A, W_IDX), jnp.int32),
        pltpu.VMEM((ROWS_A, W_IDX), jnp.int32),
        pltpu.VMEM((CHUNK_A, QW), jnp.float32),
        pltpu.VMEM((W_IDX, 8), jnp.float32),
        pltpu.VMEM_SHARED((N, QW), jnp.float32),
        pltpu.VMEM_SHARED((N, 8), jnp.float32),
        pltpu.SemaphoreType.DMA,
    ),
    compiler_params=pltpu.CompilerParams(use_tc_tiling_on_sc=False),
)
def _sc_agg1(xq0, xq1, xq2, xq3, src3, dst3, zq, z8, ones8, agg_out, cnt_out,
             srcb, dstb, rows, onesv, acc, cacc, sem):
    cid = lax.axis_index("c")
    sid = lax.axis_index("s")

    @pl.when(sid < NSI)
    def _():
        pltpu.sync_copy(zq.at[pl.ds(sid * RPS, RPS)],
                        acc.at[pl.ds(sid * RPS, RPS)])
        pltpu.sync_copy(z8.at[pl.ds(sid * RPS, RPS)],
                        cacc.at[pl.ds(sid * RPS, RPS)])

    pltpu.sync_copy(ones8, onesv)
    plsc.subcore_barrier()

    blk0 = sid * ITERS_A

    def gather(tbl):
        cps = [pltpu.async_copy(tbl.at[srcb.at[j]],
                                rows.at[pl.ds(j * W_IDX, W_IDX)], sem)
               for j in range(ROWS_A)]
        for cp in cps:
            cp.wait()

    for phase, (t0, t1) in enumerate(((xq0, xq2), (xq1, xq3))):

        def step(it, carry):
            blk = blk0 + it
            pltpu.sync_copy(src3.at[blk], srcb)
            pltpu.sync_copy(dst3.at[blk], dstb)

            @pl.when(cid == 0)
            def _():
                gather(t0)

            @pl.when(cid == 1)
            def _():
                gather(t1)

            for j in range(ROWS_A):
                pltpu.sync_copy(rows.at[pl.ds(j * W_IDX, W_IDX)],
                                acc.at[dstb.at[j]], add=True)

            if phase == 0:
                @pl.when(cid == 0)
                def _():
                    for j in range(ROWS_A):
                        pltpu.sync_copy(onesv, cacc.at[dstb.at[j]], add=True)

            return carry

        lax.fori_loop(0, ITERS_A, step, 0)
        plsc.subcore_barrier()

        qi = cid * 2 + phase

        @pl.when(sid < NSI)
        def _():
            pltpu.sync_copy(acc.at[pl.ds(sid * RPS, RPS)],
                            agg_out.at[qi, pl.ds(sid * RPS, RPS)])
            if phase == 0:
                pltpu.sync_copy(zq.at[pl.ds(sid * RPS, RPS)],
                                acc.at[pl.ds(sid * RPS, RPS)])

        if phase == 0:
            plsc.subcore_barrier()

    @pl.when(jnp.logical_and(cid == 0, sid < NSI))
    def _():
        pltpu.sync_copy(cacc.at[pl.ds(sid * RPS, RPS)],
                        cnt_out.at[pl.ds(sid * RPS, RPS)])


@functools.partial(
    pl.kernel,
    out_type=jax.ShapeDtypeStruct((NC, N, TPAD), jnp.float32),
    mesh=_mesh,
    scratch_types=(
        pltpu.VMEM((ROWS_B, W_IDX), jnp.int32),
        pltpu.VMEM((ROWS_B, W_IDX), jnp.int32),
        pltpu.VMEM((CHUNK_B, TPAD), jnp.float32),
        pltpu.VMEM_SHARED((N, TPAD), jnp.float32),
        pltpu.SemaphoreType.DMA,
    ),
    compiler_params=pltpu.CompilerParams(use_tc_tiling_on_sc=False),
)
def _sc_agg2(t, src3, dst3, z64, part_out, srcb, dstb, rows, acc, sem):
    cid = lax.axis_index("c")
    sid = lax.axis_index("s")

    @pl.when(sid < NSI)
    def _():
        pltpu.sync_copy(z64.at[pl.ds(sid * RPS, RPS)],
                        acc.at[pl.ds(sid * RPS, RPS)])

    plsc.subcore_barrier()

    wid = cid * NS + sid
    blk0 = wid * ITERS_B

    def step(it, carry):
        blk = blk0 + it
        pltpu.sync_copy(src3.at[blk], srcb)
        pltpu.sync_copy(dst3.at[blk], dstb)
        cps = [pltpu.async_copy(t.at[srcb.at[j]],
                                rows.at[pl.ds(j * W_IDX, W_IDX)], sem)
               for j in range(ROWS_B)]
        for cp in cps:
            cp.wait()
        for j in range(ROWS_B):
            pltpu.sync_copy(rows.at[pl.ds(j * W_IDX, W_IDX)],
                            acc.at[dstb.at[j]], add=True)
        return carry

    lax.fori_loop(0, ITERS_B, step, 0)
    plsc.subcore_barrier()

    @pl.when(sid < NSI)
    def _():
        pltpu.sync_copy(acc.at[pl.ds(sid * RPS, RPS)],
                        part_out.at[cid, pl.ds(sid * RPS, RPS)])


R_BLK = 2000
GRID = N // R_BLK


def _tc1_body(agg_ref, cnt_ref, x_ref, wl_ref, wr_ref, b1_ref,
              h_ref, st_ref):
    i = pl.program_id(0)
    inv = 1.0 / jnp.maximum(cnt_ref[:, 0:1], 1.0)
    h = (jnp.dot(x_ref[...], wr_ref[...], preferred_element_type=jnp.float32)
         + b1_ref[...])
    for q in range(NQ):
        h += jnp.dot(agg_ref[q] * inv, wl_ref[q],
                     preferred_element_type=jnp.float32)
    h_ref[...] = h

    @pl.when(i == 0)
    def _():
        st_ref[...] = jnp.zeros_like(st_ref)

    st_ref[0:1, :] += jnp.sum(h, axis=0, keepdims=True)
    st_ref[1:2, :] += jnp.sum(h * h, axis=0, keepdims=True)


def _tc1(agg, cnt8, x, wl2, wr, b1r):
    return pl.pallas_call(
        _tc1_body,
        grid=(GRID,),
        in_specs=[
            pl.BlockSpec((NQ, R_BLK, QW), lambda i: (0, i, 0)),
            pl.BlockSpec((R_BLK, 8), lambda i: (i, 0)),
            pl.BlockSpec((R_BLK, D), lambda i: (i, 0)),
            pl.BlockSpec((NQ, QW, HID), lambda i: (0, 0, 0)),
            pl.BlockSpec((D, HID), lambda i: (0, 0)),
            pl.BlockSpec((1, HID), lambda i: (0, 0)),
        ],
        out_specs=[
            pl.BlockSpec((R_BLK, HID), lambda i: (i, 0)),
            pl.BlockSpec((8, HID), lambda i: (0, 0)),
        ],
        out_shape=[
            jax.ShapeDtypeStruct((N, HID), jnp.float32),
            jax.ShapeDtypeStruct((8, HID), jnp.float32),
        ],
        compiler_params=pltpu.CompilerParams(
            dimension_semantics=("arbitrary",)),
    )(agg, cnt8, x, wl2, wr, b1r)


def _tc2_body(h_ref, st_ref, g_ref, bt_ref, wl_ref, wr_ref, b2_ref,
              t_ref, r_ref):
    mu = st_ref[0:1, :] * (1.0 / N)
    var = st_ref[1:2, :] * (1.0 / N) - mu * mu
    scale = g_ref[...] * lax.rsqrt(var + 1e-5)
    h = jnp.maximum((h_ref[...] - mu) * scale + bt_ref[...], 0.0)
    t_ref[...] = jnp.dot(h, wl_ref[...], preferred_element_type=jnp.float32)
    r_ref[...] = jnp.dot(h, wr_ref[...],
                         preferred_element_type=jnp.float32) + b2_ref[...]


def _tc2(hpre, stats, gr, br, wl, wr, b2r):
    return pl.pallas_call(
        _tc2_body,
        grid=(GRID,),
        in_specs=[
            pl.BlockSpec((R_BLK, HID), lambda i: (i, 0)),
            pl.BlockSpec((8, HID), lambda i: (0, 0)),
            pl.BlockSpec((1, HID), lambda i: (0, 0)),
            pl.BlockSpec((1, HID), lambda i: (0, 0)),
            pl.BlockSpec((HID, TPAD), lambda i: (0, 0)),
            pl.BlockSpec((HID, TPAD), lambda i: (0, 0)),
            pl.BlockSpec((1, TPAD), lambda i: (0, 0)),
        ],
        out_specs=[
            pl.BlockSpec((R_BLK, TPAD), lambda i: (i, 0)),
            pl.BlockSpec((R_BLK, TPAD), lambda i: (i, 0)),
        ],
        out_shape=[
            jax.ShapeDtypeStruct((N, TPAD), jnp.float32),
            jax.ShapeDtypeStruct((N, TPAD), jnp.float32),
        ],
        compiler_params=pltpu.CompilerParams(
            dimension_semantics=("arbitrary",)),
    )(hpre, stats, gr, br, wl, wr, b2r)


def _tc3_body(part_ref, cnt_ref, r_ref, o_ref):
    inv = 1.0 / jnp.maximum(cnt_ref[:, 0:1], 1.0)
    z = (part_ref[0] + part_ref[1]) * inv + r_ref[...]
    col = lax.broadcasted_iota(jnp.int32, (R_BLK, TPAD), 1)
    valid = col < NCLS
    zm = jnp.where(valid, z, -jnp.inf)
    m = jnp.max(zm, axis=1, keepdims=True)
    e = jnp.where(valid, jnp.exp(z - m), 0.0)
    lse = jnp.log(jnp.sum(e, axis=1, keepdims=True))
    o_ref[...] = z - m - lse


def _tc3(part, cnt8, r):
    return pl.pallas_call(
        _tc3_body,
        grid=(GRID,),
        in_specs=[
            pl.BlockSpec((NC, R_BLK, TPAD), lambda i: (0, i, 0)),
            pl.BlockSpec((R_BLK, 8), lambda i: (i, 0)),
            pl.BlockSpec((R_BLK, TPAD), lambda i: (i, 0)),
        ],
        out_specs=pl.BlockSpec((R_BLK, TPAD), lambda i: (i, 0)),
        out_shape=jax.ShapeDtypeStruct((N, TPAD), jnp.float32),
        compiler_params=pltpu.CompilerParams(
            dimension_semantics=("arbitrary",)),
    )(part, cnt8, r)


def kernel(x, edge_index, W1_l, W1_r, b1, gamma, beta, W2_l, W2_r, b2):
    ei = edge_index.astype(jnp.int32)
    src3a = ei[0].reshape(E // CHUNK_A, ROWS_A, W_IDX)
    dst3a = ei[1].reshape(E // CHUNK_A, ROWS_A, W_IDX)
    src3b = ei[0].reshape(E // CHUNK_B, ROWS_B, W_IDX)
    dst3b = ei[1].reshape(E // CHUNK_B, ROWS_B, W_IDX)
    xq = [x[:, q * QW:(q + 1) * QW] for q in range(NQ)]

    z64 = jnp.zeros((N, TPAD), jnp.float32)
    z8 = jnp.zeros((N, 8), jnp.float32)
    ones8 = jnp.ones((W_IDX, 8), jnp.float32)

    agg, cnt8 = _sc_agg1(xq[0], xq[1], xq[2], xq[3], src3a, dst3a,
                         z64, z8, ones8)

    wl4 = W1_l.reshape(NQ, QW, HID)
    hpre, stats = _tc1(agg, cnt8, x, wl4, W1_r, b1.reshape(1, HID))

    wl = jnp.pad(W2_l, ((0, 0), (0, TPAD - NCLS)))
    wr = jnp.pad(W2_r, ((0, 0), (0, TPAD - NCLS)))
    b2r = jnp.pad(b2, (0, TPAD - NCLS)).reshape(1, TPAD)
    t, r = _tc2(hpre, stats, gamma.reshape(1, HID), beta.reshape(1, HID),
                wl, wr, b2r)

    part = _sc_agg2(t, src3b, dst3b, z64)
    out = _tc3(part, cnt8, r)
    return out[:, :NCLS]


# speedup vs baseline: 8.1907x; 1.2803x over previous
import functools

import jax
import jax.numpy as jnp
from jax import lax
from jax.experimental import pallas as pl
from jax.experimental.pallas import tpu as pltpu
from jax.experimental.pallas import tpu_sc as plsc

N = 10000
E = 160000
D = 256
HID = 256
NCLS = 40
QW = 64
NQ = D // QW
TPAD = 64

NC, NS = 2, 16
W_IDX = 100
ROWS = 5
CHUNK = W_IDX * ROWS
NCH_A = E // NS // CHUNK
NCH_B = E // (NC * NS) // CHUNK
RPS = 1000
NSI = N // RPS

_mesh = plsc.VectorSubcoreMesh(core_axis_name="c", subcore_axis_name="s")


@functools.partial(
    pl.kernel,
    out_type=(
        jax.ShapeDtypeStruct((NQ, N, QW), jnp.float32),
        jax.ShapeDtypeStruct((NC, N, 8), jnp.float32),
    ),
    mesh=_mesh,
    scratch_types=(
        pltpu.VMEM((2, ROWS, W_IDX), jnp.int32),
        pltpu.VMEM((2, ROWS, W_IDX), jnp.int32),
        pltpu.VMEM((2, CHUNK, QW), jnp.float32),
        pltpu.VMEM((W_IDX, 8), jnp.float32),
        pltpu.VMEM_SHARED((N, QW), jnp.float32),
        pltpu.VMEM_SHARED((N, 8), jnp.float32),
        pltpu.SemaphoreType.DMA,
        pltpu.SemaphoreType.DMA,
    ),
    compiler_params=pltpu.CompilerParams(use_tc_tiling_on_sc=False),
)
def _sc_agg1(xq0, xq1, xq2, xq3, src3, dst3, zq, z8, ones8, agg_out, cnt_out,
             srcb, dstb, rows, onesv, acc, cacc, sem0, sem1):
    cid = lax.axis_index("c")
    sid = lax.axis_index("s")
    sems = (sem0, sem1)

    @pl.when(sid < NSI)
    def _():
        pltpu.sync_copy(zq.at[pl.ds(sid * RPS, RPS)],
                        acc.at[pl.ds(sid * RPS, RPS)])
        pltpu.sync_copy(z8.at[pl.ds(sid * RPS, RPS)],
                        cacc.at[pl.ds(sid * RPS, RPS)])

    pltpu.sync_copy(ones8, onesv)
    plsc.subcore_barrier()

    blk0 = sid * NCH_A

    def start(b, blk, tbl):
        pltpu.sync_copy(src3.at[blk], srcb.at[b])
        pltpu.sync_copy(dst3.at[blk], dstb.at[b])
        for j in range(ROWS):
            pltpu.async_copy(tbl.at[srcb.at[b, j]],
                             rows.at[b, pl.ds(j * W_IDX, W_IDX)], sems[b])

    def drain(b, tbl):
        for j in range(ROWS):
            pltpu.make_async_copy(tbl.at[srcb.at[b, j]],
                                  rows.at[b, pl.ds(j * W_IDX, W_IDX)],
                                  sems[b]).wait()

    def run(tbl, phase):
        for b in range(2):
            start(b, blk0 + b, tbl)

        def step(g2, carry):
            for b in range(2):
                chunk = 2 * g2 + b
                drain(b, tbl)
                for j in range(ROWS):
                    pltpu.sync_copy(rows.at[b, pl.ds(j * W_IDX, W_IDX)],
                                    acc.at[dstb.at[b, j]], add=True)
                if phase == 0:
                    @pl.when(cid == b)
                    def _():
                        for j in range(ROWS):
                            pltpu.sync_copy(onesv, cacc.at[dstb.at[b, j]],
                                            add=True)

                @pl.when(chunk + 2 < NCH_A)
                def _():
                    start(b, blk0 + chunk + 2, tbl)
            return carry

        lax.fori_loop(0, NCH_A // 2, step, 0)

    for phase, (t0, t1) in enumerate(((xq0, xq2), (xq1, xq3))):

        @pl.when(cid == 0)
        def _():
            run(t0, phase)

        @pl.when(cid == 1)
        def _():
            run(t1, phase)

        plsc.subcore_barrier()

        qi = cid * 2 + phase

        @pl.when(sid < NSI)
        def _():
            pltpu.sync_copy(acc.at[pl.ds(sid * RPS, RPS)],
                            agg_out.at[qi, pl.ds(sid * RPS, RPS)])
            if phase == 0:
                pltpu.sync_copy(zq.at[pl.ds(sid * RPS, RPS)],
                                acc.at[pl.ds(sid * RPS, RPS)])

        if phase == 0:
            plsc.subcore_barrier()

    @pl.when(sid < NSI)
    def _():
        pltpu.sync_copy(cacc.at[pl.ds(sid * RPS, RPS)],
                        cnt_out.at[cid, pl.ds(sid * RPS, RPS)])


@functools.partial(
    pl.kernel,
    out_type=jax.ShapeDtypeStruct((NC, N, TPAD), jnp.float32),
    mesh=_mesh,
    scratch_types=(
        pltpu.VMEM((2, ROWS, W_IDX), jnp.int32),
        pltpu.VMEM((2, ROWS, W_IDX), jnp.int32),
        pltpu.VMEM((2, CHUNK, TPAD), jnp.float32),
        pltpu.VMEM_SHARED((N, TPAD), jnp.float32),
        pltpu.SemaphoreType.DMA,
        pltpu.SemaphoreType.DMA,
    ),
    compiler_params=pltpu.CompilerParams(use_tc_tiling_on_sc=False),
)
def _sc_agg2(t, src3, dst3, z64, part_out, srcb, dstb, rows, acc, sem0, sem1):
    cid = lax.axis_index("c")
    sid = lax.axis_index("s")
    sems = (sem0, sem1)

    @pl.when(sid < NSI)
    def _():
        pltpu.sync_copy(z64.at[pl.ds(sid * RPS, RPS)],
                        acc.at[pl.ds(sid * RPS, RPS)])

    plsc.subcore_barrier()

    wid = cid * NS + sid
    blk0 = wid * NCH_B

    def start(b, blk):
        pltpu.sync_copy(src3.at[blk], srcb.at[b])
        pltpu.sync_copy(dst3.at[blk], dstb.at[b])
        for j in range(ROWS):
            pltpu.async_copy(t.at[srcb.at[b, j]],
                             rows.at[b, pl.ds(j * W_IDX, W_IDX)], sems[b])

    for b in range(2):
        start(b, blk0 + b)

    def step(g2, carry):
        for b in range(2):
            chunk = 2 * g2 + b
            for j in range(ROWS):
                pltpu.make_async_copy(t.at[srcb.at[b, j]],
                                      rows.at[b, pl.ds(j * W_IDX, W_IDX)],
                                      sems[b]).wait()
            for j in range(ROWS):
                pltpu.sync_copy(rows.at[b, pl.ds(j * W_IDX, W_IDX)],
                                acc.at[dstb.at[b, j]], add=True)

            @pl.when(chunk + 2 < NCH_B)
            def _():
                start(b, blk0 + chunk + 2)
        return carry

    lax.fori_loop(0, NCH_B // 2, step, 0)
    plsc.subcore_barrier()

    @pl.when(sid < NSI)
    def _():
        pltpu.sync_copy(acc.at[pl.ds(sid * RPS, RPS)],
                        part_out.at[cid, pl.ds(sid * RPS, RPS)])


R_BLK = 2000
GRID = N // R_BLK


def _tc0_body(x_ref, wr_ref, b1_ref, hr_ref):
    hr_ref[...] = (jnp.dot(x_ref[...], wr_ref[...],
                           preferred_element_type=jnp.float32) + b1_ref[...])


def _tc0(x, wr, b1r):
    return pl.pallas_call(
        _tc0_body,
        grid=(GRID,),
        in_specs=[
            pl.BlockSpec((R_BLK, D), lambda i: (i, 0)),
            pl.BlockSpec((D, HID), lambda i: (0, 0)),
            pl.BlockSpec((1, HID), lambda i: (0, 0)),
        ],
        out_specs=pl.BlockSpec((R_BLK, HID), lambda i: (i, 0)),
        out_shape=jax.ShapeDtypeStruct((N, HID), jnp.float32),
        compiler_params=pltpu.CompilerParams(
            dimension_semantics=("arbitrary",)),
    )(x, wr, b1r)


def _tc1_body(agg_ref, cnt_ref, hr_ref, wl_ref, h_ref, st_ref):
    i = pl.program_id(0)
    cnt = cnt_ref[0, :, 0:1] + cnt_ref[1, :, 0:1]
    inv = 1.0 / jnp.maximum(cnt, 1.0)
    h = hr_ref[...]
    for q in range(NQ):
        h += jnp.dot(agg_ref[q] * inv, wl_ref[q],
                     preferred_element_type=jnp.float32)
    h_ref[...] = h

    @pl.when(i == 0)
    def _():
        st_ref[...] = jnp.zeros_like(st_ref)

    st_ref[0:1, :] += jnp.sum(h, axis=0, keepdims=True)
    st_ref[1:2, :] += jnp.sum(h * h, axis=0, keepdims=True)


def _tc1(agg, cnt8, hr, wl4):
    return pl.pallas_call(
        _tc1_body,
        grid=(GRID,),
        in_specs=[
            pl.BlockSpec((NQ, R_BLK, QW), lambda i: (0, i, 0)),
            pl.BlockSpec((NC, R_BLK, 8), lambda i: (0, i, 0)),
            pl.BlockSpec((R_BLK, HID), lambda i: (i, 0)),
            pl.BlockSpec((NQ, QW, HID), lambda i: (0, 0, 0)),
        ],
        out_specs=[
            pl.BlockSpec((R_BLK, HID), lambda i: (i, 0)),
            pl.BlockSpec((8, HID), lambda i: (0, 0)),
        ],
        out_shape=[
            jax.ShapeDtypeStruct((N, HID), jnp.float32),
            jax.ShapeDtypeStruct((8, HID), jnp.float32),
        ],
        compiler_params=pltpu.CompilerParams(
            dimension_semantics=("arbitrary",)),
    )(agg, cnt8, hr, wl4)


def _tc2_body(h_ref, st_ref, g_ref, bt_ref, wl_ref, wr_ref, b2_ref,
              t_ref, r_ref):
    mu = st_ref[0:1, :] * (1.0 / N)
    var = st_ref[1:2, :] * (1.0 / N) - mu * mu
    scale = g_ref[...] * lax.rsqrt(var + 1e-5)
    h = jnp.maximum((h_ref[...] - mu) * scale + bt_ref[...], 0.0)
    t_ref[...] = jnp.dot(h, wl_ref[...], preferred_element_type=jnp.float32)
    r_ref[...] = jnp.dot(h, wr_ref[...],
                         preferred_element_type=jnp.float32) + b2_ref[...]


def _tc2(hpre, stats, gr, br, wl, wr, b2r):
    return pl.pallas_call(
        _tc2_body,
        grid=(GRID,),
        in_specs=[
            pl.BlockSpec((R_BLK, HID), lambda i: (i, 0)),
            pl.BlockSpec((8, HID), lambda i: (0, 0)),
            pl.BlockSpec((1, HID), lambda i: (0, 0)),
            pl.BlockSpec((1, HID), lambda i: (0, 0)),
            pl.BlockSpec((HID, TPAD), lambda i: (0, 0)),
            pl.BlockSpec((HID, TPAD), lambda i: (0, 0)),
            pl.BlockSpec((1, TPAD), lambda i: (0, 0)),
        ],
        out_specs=[
            pl.BlockSpec((R_BLK, TPAD), lambda i: (i, 0)),
            pl.BlockSpec((R_BLK, TPAD), lambda i: (i, 0)),
        ],
        out_shape=[
            jax.ShapeDtypeStruct((N, TPAD), jnp.float32),
            jax.ShapeDtypeStruct((N, TPAD), jnp.float32),
        ],
        compiler_params=pltpu.CompilerParams(
            dimension_semantics=("arbitrary",)),
    )(hpre, stats, gr, br, wl, wr, b2r)


def _tc3_body(part_ref, cnt_ref, r_ref, o_ref):
    cnt = cnt_ref[0, :, 0:1] + cnt_ref[1, :, 0:1]
    inv = 1.0 / jnp.maximum(cnt, 1.0)
    z = (part_ref[0] + part_ref[1]) * inv + r_ref[...]
    col = lax.broadcasted_iota(jnp.int32, (R_BLK, TPAD), 1)
    valid = col < NCLS
    zm = jnp.where(valid, z, -jnp.inf)
    m = jnp.max(zm, axis=1, keepdims=True)
    e = jnp.where(valid, jnp.exp(z - m), 0.0)
    lse = jnp.log(jnp.sum(e, axis=1, keepdims=True))
    o_ref[...] = z - m - lse


def _tc3(part, cnt8, r):
    return pl.pallas_call(
        _tc3_body,
        grid=(GRID,),
        in_specs=[
            pl.BlockSpec((NC, R_BLK, TPAD), lambda i: (0, i, 0)),
            pl.BlockSpec((NC, R_BLK, 8), lambda i: (0, i, 0)),
            pl.BlockSpec((R_BLK, TPAD), lambda i: (i, 0)),
        ],
        out_specs=pl.BlockSpec((R_BLK, TPAD), lambda i: (i, 0)),
        out_shape=jax.ShapeDtypeStruct((N, TPAD), jnp.float32),
        compiler_params=pltpu.CompilerParams(
            dimension_semantics=("arbitrary",)),
    )(part, cnt8, r)


def kernel(x, edge_index, W1_l, W1_r, b1, gamma, beta, W2_l, W2_r, b2):
    ei = edge_index.astype(jnp.int32)
    src3 = ei[0].reshape(E // CHUNK, ROWS, W_IDX)
    dst3 = ei[1].reshape(E // CHUNK, ROWS, W_IDX)
    xq = [x[:, q * QW:(q + 1) * QW] for q in range(NQ)]

    z64 = jnp.zeros((N, TPAD), jnp.float32)
    z8 = jnp.zeros((N, 8), jnp.float32)
    ones8 = jnp.ones((W_IDX, 8), jnp.float32)

    agg, cnt8 = _sc_agg1(xq[0], xq[1], xq[2], xq[3], src3, dst3,
                         z64, z8, ones8)
    hr = _tc0(x, W1_r, b1.reshape(1, HID))

    wl4 = W1_l.reshape(NQ, QW, HID)
    hpre, stats = _tc1(agg, cnt8, hr, wl4)

    wl = jnp.pad(W2_l, ((0, 0), (0, TPAD - NCLS)))
    wr = jnp.pad(W2_r, ((0, 0), (0, TPAD - NCLS)))
    b2r = jnp.pad(b2, (0, TPAD - NCLS)).reshape(1, TPAD)
    t, r = _tc2(hpre, stats, gamma.reshape(1, HID), beta.reshape(1, HID),
                wl, wr, b2r)

    part = _sc_agg2(t, src3, dst3, z64)
    out = _tc3(part, cnt8, r)
    return out[:, :NCLS]
